# baseline (speedup 1.0000x reference)
import jax
import jax.numpy as jnp
from jax.experimental import pallas as pl
from jax.experimental.pallas import tpu as pltpu

_TM = 512
_TKW = 1024
_NQ = 4


def _make_kernel(n_cast, n_comp, cq):
    n_q = _NQ

    def _body(x_ref, w_ref, b_ref, o_ref, wbf_ref):
        s = pl.program_id(0)

        @pl.when(s < n_cast)
        def _cast_first():
            wbf_ref[0, pl.ds(s * _TKW, _TKW), :] = w_ref[...].astype(
                jnp.bfloat16
            )

        @pl.when(s >= n_cast)
        def _compute_and_cast():
            t = s - n_cast
            q = t // n_comp

            m = t - q * n_comp
            @pl.when(jnp.logical_and(q < n_q - 1, m % 2 == 0))
            def _cast_next():
                r = m // 2
                wbf_ref[(q + 1) % 2, pl.ds(r * _TKW, _TKW), :] = w_ref[
                    ...
                ].astype(jnp.bfloat16)

            x_bf = x_ref[...].astype(jnp.bfloat16)
            o_ref[...] = (
                jnp.dot(
                    x_bf, wbf_ref[q % 2], preferred_element_type=jnp.float32
                )
                + b_ref[...]
            )

    return _body


def kernel(x, w_t, bias):
    B, D = x.shape
    D2, C = w_t.shape
    assert D2 == D
    cq = C // _NQ
    n_cast = D // _TKW
    n_comp = B // _TM
    assert n_comp == 2 * n_cast
    grid = (n_cast + _NQ * n_comp,)

    b2 = bias.astype(jnp.float32).reshape(1, C)

    def x_idx(s):
        t = jnp.maximum(s - n_cast, 0)
        return (t % n_comp, 0)

    def w_idx(s):
        t = jnp.maximum(s - n_cast, 0)
        q_next = jnp.minimum(t // n_comp + 1, _NQ - 1)
        quarter = jnp.where(s < n_cast, 0, q_next)
        r = jnp.where(s < n_cast, s, (t % n_comp) // 2)
        r = jnp.where(t // n_comp >= _NQ - 1, n_cast - 1, r)
        return (r, quarter)

    def o_idx(s):
        t = jnp.maximum(s - n_cast, 0)
        return (t % n_comp, t // n_comp)

    def b_idx(s):
        t = jnp.maximum(s - n_cast, 0)
        return (0, t // n_comp)

    return pl.pallas_call(
        _make_kernel(n_cast, n_comp, cq),
        out_shape=jax.ShapeDtypeStruct((B, C), jnp.float32),
        grid=grid,
        in_specs=[
            pl.BlockSpec((_TM, D), x_idx),
            pl.BlockSpec((_TKW, cq), w_idx),
            pl.BlockSpec((1, cq), b_idx),
        ],
        out_specs=pl.BlockSpec((_TM, cq), o_idx),
        scratch_shapes=[pltpu.VMEM((2, D, cq), jnp.bfloat16)],
        compiler_params=pltpu.CompilerParams(
            dimension_semantics=("arbitrary",),
            vmem_limit_bytes=64 * 1024 * 1024,
        ),
    )(x, w_t, b2)


# speedup vs baseline: 1.0410x; 1.0410x over previous
import jax
import jax.numpy as jnp
from jax.experimental import pallas as pl
from jax.experimental.pallas import tpu as pltpu

_TM = 512
_TKW = 512
_NQ = 4


def _make_kernel(n_cast, n_comp, cq):
    n_q = _NQ

    def _body(x_ref, w_ref, b_ref, o_ref, wbf_ref):
        s = pl.program_id(0)

        @pl.when(s < n_cast)
        def _cast_first():
            wbf_ref[0, pl.ds(s * _TKW, _TKW), :] = w_ref[...].astype(
                jnp.bfloat16
            )

        @pl.when(s >= n_cast)
        def _compute_and_cast():
            t = s - n_cast
            q = t // n_comp

            @pl.when(q < n_q - 1)
            def _cast_next():
                r = t - q * n_comp
                wbf_ref[(q + 1) % 2, pl.ds(r * _TKW, _TKW), :] = w_ref[
                    ...
                ].astype(jnp.bfloat16)

            x_bf = x_ref[...].astype(jnp.bfloat16)
            o_ref[...] = (
                jnp.dot(
                    x_bf, wbf_ref[q % 2], preferred_element_type=jnp.float32
                )
                + b_ref[...]
            )

    return _body


def kernel(x, w_t, bias):
    B, D = x.shape
    D2, C = w_t.shape
    assert D2 == D
    cq = C // _NQ
    n_cast = D // _TKW
    n_comp = B // _TM
    assert n_cast == n_comp
    grid = (n_cast + _NQ * n_comp,)

    b2 = bias.astype(jnp.float32).reshape(1, C)

    def x_idx(s):
        t = jnp.maximum(s - n_cast, 0)
        return (t % n_comp, 0)

    def w_idx(s):
        t = jnp.maximum(s - n_cast, 0)
        q_next = jnp.minimum(t // n_comp + 1, _NQ - 1)
        quarter = jnp.where(s < n_cast, 0, q_next)
        r = jnp.where(s < n_cast, s, t % n_comp)
        r = jnp.where(t // n_comp >= _NQ - 1, n_cast - 1, r)
        return (r, quarter)

    def o_idx(s):
        t = jnp.maximum(s - n_cast, 0)
        return (t % n_comp, t // n_comp)

    def b_idx(s):
        t = jnp.maximum(s - n_cast, 0)
        return (0, t // n_comp)

    return pl.pallas_call(
        _make_kernel(n_cast, n_comp, cq),
        out_shape=jax.ShapeDtypeStruct((B, C), jnp.float32),
        grid=grid,
        in_specs=[
            pl.BlockSpec((_TM, D), x_idx),
            pl.BlockSpec((_TKW, cq), w_idx),
            pl.BlockSpec((1, cq), b_idx),
        ],
        out_specs=pl.BlockSpec((_TM, cq), o_idx),
        scratch_shapes=[pltpu.VMEM((2, D, cq), jnp.bfloat16)],
        compiler_params=pltpu.CompilerParams(
            dimension_semantics=("arbitrary",),
            vmem_limit_bytes=64 * 1024 * 1024,
        ),
    )(x, w_t, b2)
